# baseline (speedup 1.0000x reference)
import functools

import jax
import jax.numpy as jnp
from jax import lax
from jax.experimental import pallas as pl
from jax.experimental.pallas import tpu as pltpu
from jax.experimental.pallas import tpu_sc as plsc

NC = 2
NS = 16
NW = NC * NS

LINE = 128
G = 80
DEPTH = 4
VREG = 16
UNROLL = 8


def _gather_kernel(B, D, table_hbm, idx_hbm, out_hbm, idx_v, *bufs):
    rows = bufs[0:DEPTH]
    comp = bufs[DEPTH:2 * DEPTH]
    gsem = bufs[2 * DEPTH:3 * DEPTH]
    wsem = bufs[3 * DEPTH:4 * DEPTH]

    b_per_w = B // NW
    groups = b_per_w // G
    wid = lax.axis_index("s") * NC + lax.axis_index("c")
    tok_base = wid * b_per_w
    pltpu.sync_copy(idx_hbm.at[pl.ds(tok_base, b_per_w)], idx_v)

    def compact(rows_v, comp_v):
        @plsc.parallel_loop(0, G, unroll=UNROLL)
        def _(r):
            for c0 in range(0, D, VREG):
                comp_v[r, pl.ds(c0, VREG)] = rows_v[r, pl.ds(c0, VREG)]

    @pl.loop(0, groups // DEPTH)
    def _(q):
        g0 = q * DEPTH
        ga = [pltpu.async_copy(
                  table_hbm.at[idx_v.at[pl.ds((g0 + i) * G, G)]],
                  rows[i], gsem[i])
              for i in range(DEPTH)]
        ws = []
        for i in range(DEPTH):
            ga[i].wait()
            compact(rows[i], comp[i])
            ws.append(pltpu.async_copy(
                comp[i],
                out_hbm.at[pl.ds(tok_base + (g0 + i) * G, G)],
                wsem[i]))
        for w in ws:
            w.wait()


def _pad_kernel(x_ref, o_ref):
    o_ref[:, : x_ref.shape[1]] = x_ref[...]


def _pad_table(table):
    V, D = table.shape
    RB = 2000
    return pl.pallas_call(
        _pad_kernel,
        grid=(V // RB,),
        in_specs=[pl.BlockSpec((RB, D), lambda i: (i, 0))],
        out_specs=pl.BlockSpec((RB, LINE), lambda i: (i, 0)),
        out_shape=jax.ShapeDtypeStruct((V, LINE), table.dtype),
    )(table)


def kernel(phonemes, table):
    S0, S1 = phonemes.shape
    B = S0 * S1
    V, D = table.shape
    idx = phonemes.reshape(B).astype(jnp.int32)
    table_pad = _pad_table(table)

    mesh = plsc.VectorSubcoreMesh(core_axis_name="c", subcore_axis_name="s")
    b_per_w = B // NW

    k = pl.kernel(
        functools.partial(_gather_kernel, B, D),
        out_type=jax.ShapeDtypeStruct((B, D), table.dtype),
        mesh=mesh,
        scratch_types=(
            [pltpu.VMEM((b_per_w,), jnp.int32)]
            + [pltpu.VMEM((G, LINE), jnp.float32)] * DEPTH
            + [pltpu.VMEM((G, D), jnp.float32)] * DEPTH
            + [pltpu.SemaphoreType.DMA] * (2 * DEPTH)
        ),
    )
    return k(table_pad, idx).reshape(S0, S1, D)
